# baseline (speedup 1.0000x reference)
import functools
import math

import jax
import jax.numpy as jnp
from jax import lax
from jax.experimental import pallas as pl
from jax.experimental.pallas import tpu as pltpu
from jax.experimental.pallas import tpu_sc as plsc

_LANES = 16


@functools.lru_cache(maxsize=None)
def _build(n_rows: int, vocab: int, d_model: int, scale: float):
    info = plsc.get_sparse_core_info()
    nc, ns = info.num_cores, info.num_subcores
    nw = nc * ns
    assert n_rows % nw == 0
    rows_per_worker = n_rows // nw

    chunk = 512
    while rows_per_worker % chunk:
        chunk //= 2
    n_chunks = rows_per_worker // chunk
    d_vecs = d_model // _LANES

    mesh = plsc.VectorSubcoreMesh(core_axis_name="c", subcore_axis_name="s")

    @functools.partial(
        pl.kernel,
        out_type=jax.ShapeDtypeStruct((n_rows, d_model), jnp.float32),
        mesh=mesh,
        scratch_types=[
            pltpu.VMEM((chunk,), jnp.int32),
            pltpu.VMEM((chunk, d_model), jnp.float32),
            pltpu.SemaphoreType.DMA,
        ],
        compiler_params=pltpu.CompilerParams(use_tc_tiling_on_sc=False),
    )
    def gather_scale(table_hbm, idx_hbm, out_hbm, idx_v, rows_v, sem):
        wid = lax.axis_index("s") * nc + lax.axis_index("c")
        base = wid * rows_per_worker

        def chunk_body(g, _):
            off = base + g * chunk
            pltpu.sync_copy(idx_hbm.at[pl.ds(off, chunk)], idx_v)
            pltpu.async_copy(table_hbm.at[idx_v], rows_v, sem).wait()

            def scale_row(r, _):
                for d in range(d_vecs):
                    sl = pl.ds(d * _LANES, _LANES)
                    rows_v[r, sl] = rows_v[r, sl] * scale
                return 0

            lax.fori_loop(0, chunk, scale_row, 0)
            pltpu.sync_copy(rows_v, out_hbm.at[pl.ds(off, chunk)])
            return 0

        lax.fori_loop(0, n_chunks, chunk_body, 0)

    return gather_scale


def kernel(token, lookup_table):
    batch, hist = token.shape
    vocab, d_model = lookup_table.shape
    scale = math.sqrt(d_model)
    idx = token.reshape(-1).astype(jnp.int32)
    fn = _build(batch * hist, vocab, d_model, scale)
    out = fn(lookup_table, idx)
    return out.reshape(batch, hist, d_model)


# speedup vs baseline: 1.1355x; 1.1355x over previous
import functools
import math

import jax
import jax.numpy as jnp
from jax import lax
from jax.experimental import pallas as pl
from jax.experimental.pallas import tpu as pltpu
from jax.experimental.pallas import tpu_sc as plsc

_LANES = 16


@functools.lru_cache(maxsize=None)
def _build(n_rows: int, vocab: int, d_model: int, scale: float):
    info = plsc.get_sparse_core_info()
    nc, ns = info.num_cores, info.num_subcores
    nw = nc * ns
    assert n_rows % nw == 0
    rows_per_worker = n_rows // nw

    nbuf = 4
    chunk = 256
    while rows_per_worker % (chunk * nbuf):
        chunk //= 2
    n_chunks = rows_per_worker // chunk
    n_groups = n_chunks // nbuf
    d_vecs = d_model // _LANES

    mesh = plsc.VectorSubcoreMesh(core_axis_name="c", subcore_axis_name="s")

    @functools.partial(
        pl.kernel,
        out_type=jax.ShapeDtypeStruct((n_rows, d_model), jnp.float32),
        mesh=mesh,
        scratch_types=[
            pltpu.VMEM((rows_per_worker,), jnp.int32),
            pltpu.VMEM((nbuf, chunk, d_model), jnp.float32),
            [pltpu.SemaphoreType.DMA] * nbuf,
            [pltpu.SemaphoreType.DMA] * nbuf,
        ],
        compiler_params=pltpu.CompilerParams(use_tc_tiling_on_sc=False),
    )
    def gather_scale(table_hbm, idx_hbm, out_hbm, idx_v, rows_v, gsems, wsems):
        wid = lax.axis_index("s") * nc + lax.axis_index("c")
        base = wid * rows_per_worker
        pltpu.sync_copy(idx_hbm.at[pl.ds(base, rows_per_worker)], idx_v)

        def gather(g, b):
            off = pl.multiple_of(g * chunk, chunk)
            return pltpu.make_async_copy(
                table_hbm.at[idx_v.at[pl.ds(off, chunk)]], rows_v.at[b], gsems[b]
            )

        def write(g, b):
            return pltpu.make_async_copy(
                rows_v.at[b], out_hbm.at[pl.ds(base + g * chunk, chunk)], wsems[b]
            )

        gather(0, 0).start()
        gather(1, 1).start()

        def group_body(g0, _):
            for b in range(nbuf):
                g = g0 * nbuf + b
                bn = (b + 2) % nbuf

                @pl.when(g >= 2)
                def _():
                    write(g - 2, bn).wait()

                @pl.when(g + 2 < n_chunks)
                def _():
                    gather(g + 2, bn).start()

                gather(g, b).wait()

                @plsc.parallel_loop(0, chunk, unroll=8)
                def _(r):
                    for d in range(d_vecs):
                        sl = pl.ds(d * _LANES, _LANES)
                        rows_v[b, r, sl] = rows_v[b, r, sl] * scale

                write(g, b).start()
            return 0

        lax.fori_loop(0, n_groups, group_body, 0)
        write(n_chunks - 2, (n_chunks - 2) % nbuf).wait()
        write(n_chunks - 1, (n_chunks - 1) % nbuf).wait()

    return gather_scale


def kernel(token, lookup_table):
    batch, hist = token.shape
    vocab, d_model = lookup_table.shape
    scale = math.sqrt(d_model)
    idx = token.reshape(-1).astype(jnp.int32)
    fn = _build(batch * hist, vocab, d_model, scale)
    out = fn(lookup_table, idx)
    return out.reshape(batch, hist, d_model)


# speedup vs baseline: 1.1361x; 1.0005x over previous
import functools
import math

import jax
import jax.numpy as jnp
from jax import lax
from jax.experimental import pallas as pl
from jax.experimental.pallas import tpu as pltpu
from jax.experimental.pallas import tpu_sc as plsc

_LANES = 16


@functools.lru_cache(maxsize=None)
def _build(batch: int, hist: int, vocab: int, d_model: int, scale: float):
    info = plsc.get_sparse_core_info()
    nc, ns = info.num_cores, info.num_subcores
    nw = nc * ns
    assert batch % nw == 0
    tr_per_worker = batch // nw
    chunk = hist
    nbuf = 4
    n_chunks = tr_per_worker
    assert n_chunks % nbuf == 0
    n_groups = n_chunks // nbuf
    d_vecs = d_model // _LANES

    mesh = plsc.VectorSubcoreMesh(core_axis_name="c", subcore_axis_name="s")

    @functools.partial(
        pl.kernel,
        out_type=jax.ShapeDtypeStruct((batch, hist, d_model), jnp.float32),
        mesh=mesh,
        scratch_types=[
            pltpu.VMEM((tr_per_worker, hist), jnp.int32),
            pltpu.VMEM((nbuf, chunk, d_model), jnp.float32),
            [pltpu.SemaphoreType.DMA] * nbuf,
            [pltpu.SemaphoreType.DMA] * nbuf,
        ],
        compiler_params=pltpu.CompilerParams(use_tc_tiling_on_sc=False),
    )
    def gather_scale(table_hbm, tok_hbm, out_hbm, idx_v, rows_v, gsems, wsems):
        wid = lax.axis_index("s") * nc + lax.axis_index("c")
        base = wid * tr_per_worker
        pltpu.sync_copy(tok_hbm.at[pl.ds(base, tr_per_worker)], idx_v)

        def gather(g, b):
            return pltpu.make_async_copy(
                table_hbm.at[idx_v.at[g]], rows_v.at[b], gsems[b]
            )

        def write(g, b):
            return pltpu.make_async_copy(
                rows_v.at[b], out_hbm.at[base + g], wsems[b]
            )

        gather(0, 0).start()
        gather(1, 1).start()

        def group_body(g0, _):
            for b in range(nbuf):
                g = g0 * nbuf + b
                bn = (b + 2) % nbuf

                @pl.when(g >= 2)
                def _():
                    write(g - 2, bn).wait()

                @pl.when(g + 2 < n_chunks)
                def _():
                    gather(g + 2, bn).start()

                gather(g, b).wait()

                @plsc.parallel_loop(0, chunk, unroll=8)
                def _(r):
                    for d in range(d_vecs):
                        sl = pl.ds(d * _LANES, _LANES)
                        rows_v[b, r, sl] = rows_v[b, r, sl] * scale

                write(g, b).start()
            return 0

        lax.fori_loop(0, n_groups, group_body, 0)
        write(n_chunks - 2, (n_chunks - 2) % nbuf).wait()
        write(n_chunks - 1, (n_chunks - 1) % nbuf).wait()

    return gather_scale


def kernel(token, lookup_table):
    batch, hist = token.shape
    vocab, d_model = lookup_table.shape
    scale = math.sqrt(d_model)
    fn = _build(batch, hist, vocab, d_model, scale)
    return fn(lookup_table, token.astype(jnp.int32))
